# baseline (speedup 1.0000x reference)
import functools

import numpy as np
import jax
import jax.numpy as jnp
from jax import lax
from jax.experimental import pallas as pl
from jax.experimental.pallas import tpu as pltpu
from jax.experimental.pallas import tpu_sc as plsc

_K = 16
_B = 128
_V = 100000
_S = 28672
_VT = _V - _S
_W = 1024
_NV = (_VT + _W - 1) // _W
_WS = 2048
_NS = _S // _WS

_ROTS = ((13, 15, 26, 6), (17, 29, 16, 24))
_TINY = np.float32(np.finfo(np.float32).tiny)
_IMAX = np.int32(2**31 - 1)


def _np_threefry2x32(k0, k1, x0, x1):
    k0 = np.uint32(k0)
    k1 = np.uint32(k1)
    ks2 = np.uint32(k0 ^ k1 ^ np.uint32(0x1BD11BDA))
    sched = ((k1, ks2), (ks2, k0), (k0, k1), (k1, ks2), (ks2, k0))
    x0 = np.uint32(x0 + k0)
    x1 = np.uint32(x1 + k1)
    for i in range(5):
        for r in _ROTS[i % 2]:
            x0 = np.uint32(x0 + x1)
            x1 = np.uint32(np.uint32(x1 << np.uint32(r)) | np.uint32(x1 >> np.uint32(32 - r)))
            x1 = np.uint32(x1 ^ x0)
        a, b = sched[i]
        x0 = np.uint32(x0 + a)
        x1 = np.uint32(x1 + b + np.uint32(i + 1))
    return x0, x1


def _fold_in_keys():
    rows = []
    with np.errstate(over="ignore"):
        for seed in (101, 202):
            for j in range(_K):
                rows.append(_np_threefry2x32(0, seed, 0, j))
    return np.asarray(rows, dtype=np.uint32)


_KEYS = _fold_in_keys()
_KEYS_FLAT = np.zeros((2 * _K, 16), np.uint32)
_KEYS_FLAT[:, 0] = _KEYS[:, 0]
_KEYS_FLAT[:, 1] = _KEYS[:, 1]
_KEYS_FLAT = _KEYS_FLAT.reshape(-1)


def _sc_body(keys_hbm, u_hbm, keys_v, rowbuf):
    pltpu.sync_copy(keys_hbm, keys_v)
    wid = lax.axis_index("s") * 2 + lax.axis_index("c")
    kv = keys_v[pl.ds(wid * 16, 16)]
    k0 = kv[0]
    k1 = kv[1]
    ks2 = k0 ^ k1 ^ np.uint32(0x1BD11BDA)
    sched = ((k1, ks2), (ks2, k0), (k0, k1), (k1, ks2), (ks2, k0))
    lane = lax.iota(jnp.int32, 16).astype(jnp.uint32)

    def row(b, carry):
        def chunk(i, carry2):
            base = (b * _V + _VT + i * 16).astype(jnp.uint32)
            flat = lane + base
            x1 = flat + k1
            x0 = x1 + k0
            first = True
            for ri in range(5):
                for r in _ROTS[ri % 2]:
                    if first:
                        first = False
                    else:
                        x0 = x0 + x1
                    x1 = ((x1 << r) | (x1 >> (32 - r))) ^ x0
                a, bb = sched[ri]
                x0 = x0 + a
                x1 = x1 + (bb + np.uint32(ri + 1))
            bits = x0 ^ x1
            fb = (bits >> 9) | np.uint32(0x3F800000)
            u = jnp.maximum(
                lax.bitcast_convert_type(fb, jnp.float32) - np.float32(1.0), _TINY)
            rowbuf[pl.ds(i * 16, 16)] = u
            return carry2

        lax.fori_loop(0, _S // 16, chunk, 0)
        pltpu.sync_copy(rowbuf, u_hbm.at[wid, b])
        return carry

    lax.fori_loop(0, _B, row, 0)


def _sc_uniforms(keys):
    return pl.kernel(
        _sc_body,
        out_type=jax.ShapeDtypeStruct((2 * _K, _B, _S), jnp.float32),
        mesh=plsc.VectorSubcoreMesh(core_axis_name="c", subcore_axis_name="s"),
        scratch_types=[
            pltpu.VMEM((2 * _K * 16,), jnp.uint32),
            pltpu.VMEM((_S,), jnp.float32),
        ],
    )(keys)


def _body1(keys_ref, a_ref, b_ref, lmf_ref, laf_ref, xm_ref, rm_ref, ci_ref):
    v = pl.program_id(0)
    v0 = (v * _W).astype(jnp.uint32)

    col = lax.broadcasted_iota(jnp.uint32, (_B, _W), 1) + v0
    rowv = lax.broadcasted_iota(jnp.uint32, (_B, _W), 0) * np.uint32(_V)
    flat = rowv + col
    valid = col < np.uint32(_VT)
    coli = lax.bitcast_convert_type(col, jnp.int32)

    @pl.when(v == 0)
    def _():
        rm_ref[...] = jnp.full((2 * _K, _B, _W), -jnp.inf, jnp.float32)
        ci_ref[...] = jnp.zeros((2 * _K, _B, _W), jnp.int32)

    xm_ref[0] = jnp.where(valid, a_ref[...], -jnp.inf)
    xm_ref[1] = jnp.where(valid, b_ref[...], -jnp.inf)

    def jbody(t, carry):
        k0 = keys_ref[t, 0]
        k1 = keys_ref[t, 1]
        ks2 = k0 ^ k1 ^ np.uint32(0x1BD11BDA)
        sched = ((k1, ks2), (ks2, k0), (k0, k1), (k1, ks2), (ks2, k0))
        x1 = flat + k1
        x0 = x1 + k0
        first = True
        for i in range(5):
            for r in _ROTS[i % 2]:
                if first:
                    first = False
                else:
                    x0 = x0 + x1
                x1 = ((x1 << r) | (x1 >> (32 - r))) ^ x0
            a, b = sched[i]
            x0 = x0 + a
            x1 = x1 + (b + np.uint32(i + 1))
        bits = x0 ^ x1
        fb = (bits >> 9) | np.uint32(0x3F800000)
        u = jnp.maximum(lax.bitcast_convert_type(fb, jnp.float32) - np.float32(1.0), _TINY)
        g = -jnp.log(-jnp.log(u))
        y = xm_ref[t // _K] + g
        old = rm_ref[t]
        upd = y > old
        rm_ref[t] = jnp.maximum(y, old)
        ci_ref[t] = jnp.where(upd, coli, ci_ref[t])
        return carry

    lax.fori_loop(0, 2 * _K, jbody, 0, unroll=4)

    @pl.when(v == _NV - 1)
    def _():
        for t in range(2 * _K):
            rmj = rm_ref[t]
            lm = jnp.max(rmj, axis=1, keepdims=True)
            cand = jnp.where(rmj == lm, ci_ref[t], _IMAX)
            la = jnp.min(cand, axis=1, keepdims=True)
            lmf_ref[:, t:t + 1] = lm
            laf_ref[:, t:t + 1] = la


def _body2(gt_ref, lmf_ref, laf_ref, ua_ref, ub_ref, xa_ref, xb_ref,
           out_ref, rm_ref, ci_ref):
    j = pl.program_id(0)
    v = pl.program_id(1)
    coli = (lax.broadcasted_iota(jnp.int32, (_B, _WS), 1)
            + (np.int32(_VT) + v * _WS))

    @pl.when(v == 0)
    def _():
        rm_ref[...] = jnp.full((2, _B, _WS), -jnp.inf, jnp.float32)
        ci_ref[...] = jnp.zeros((2, _B, _WS), jnp.int32)

    for d, (u_ref, x_ref) in enumerate(((ua_ref, xa_ref), (ub_ref, xb_ref))):
        u = u_ref[0]
        g = -jnp.log(-jnp.log(u))
        y = x_ref[...] + g
        old = rm_ref[d]
        upd = y > old
        rm_ref[d] = jnp.maximum(y, old)
        ci_ref[d] = jnp.where(upd, coli, ci_ref[d])

    @pl.when(v == _NS - 1)
    def _():
        la_final = []
        for d in (0, 1):
            t = d * _K + j
            rmj = rm_ref[d]
            lm2 = jnp.max(rmj, axis=1, keepdims=True)
            cand = jnp.where(rmj == lm2, ci_ref[d], _IMAX)
            la2 = jnp.min(cand, axis=1, keepdims=True)
            tcol = lax.broadcasted_iota(jnp.int32, (_B, 2 * _K), 1)
            sel = tcol == t
            lm1 = jnp.max(jnp.where(sel, lmf_ref[...], -jnp.inf),
                          axis=1, keepdims=True)
            la1 = jnp.max(jnp.where(sel, laf_ref[...], 0),
                          axis=1, keepdims=True)
            use2 = lm2 > lm1
            la_final.append(jnp.where(use2, la2, la1))
        res = la_final[0] + la_final[1]
        match = (res == gt_ref[...]).astype(jnp.float32)
        fmean = jnp.sum(match) * np.float32(1.0 / _B)
        lj = -jnp.log(fmean + np.float32(1e-8)) * np.float32(1.0 / _K)
        prev = jnp.where(j == 0, np.float32(0.0), out_ref[0, 0])
        out_ref[0, 0] = prev + lj


def kernel(gt, logits_a, logits_b):
    gt2 = gt.astype(jnp.int32).reshape(_B, 1)
    u = _sc_uniforms(jnp.asarray(_KEYS_FLAT))
    lmf, laf = pl.pallas_call(
        _body1,
        grid=(_NV,),
        in_specs=[
            pl.BlockSpec(memory_space=pltpu.SMEM),
            pl.BlockSpec((_B, _W), lambda v: (0, v)),
            pl.BlockSpec((_B, _W), lambda v: (0, v)),
        ],
        out_specs=[
            pl.BlockSpec((_B, 2 * _K), lambda v: (0, 0)),
            pl.BlockSpec((_B, 2 * _K), lambda v: (0, 0)),
        ],
        out_shape=[
            jax.ShapeDtypeStruct((_B, 2 * _K), jnp.float32),
            jax.ShapeDtypeStruct((_B, 2 * _K), jnp.int32),
        ],
        scratch_shapes=[
            pltpu.VMEM((2, _B, _W), jnp.float32),
            pltpu.VMEM((2 * _K, _B, _W), jnp.float32),
            pltpu.VMEM((2 * _K, _B, _W), jnp.int32),
        ],
        compiler_params=pltpu.CompilerParams(
            dimension_semantics=("arbitrary",)),
    )(jnp.asarray(_KEYS), logits_a, logits_b)

    xa = lax.slice(logits_a, (0, _VT), (_B, _V))
    xb = lax.slice(logits_b, (0, _VT), (_B, _V))
    out = pl.pallas_call(
        _body2,
        grid=(_K, _NS),
        in_specs=[
            pl.BlockSpec((_B, 1), lambda j, v: (0, 0)),
            pl.BlockSpec((_B, 2 * _K), lambda j, v: (0, 0)),
            pl.BlockSpec((_B, 2 * _K), lambda j, v: (0, 0)),
            pl.BlockSpec((1, _B, _WS), lambda j, v: (j, 0, v)),
            pl.BlockSpec((1, _B, _WS), lambda j, v: (j + _K, 0, v)),
            pl.BlockSpec((_B, _WS), lambda j, v: (0, v)),
            pl.BlockSpec((_B, _WS), lambda j, v: (0, v)),
        ],
        out_specs=pl.BlockSpec(memory_space=pltpu.SMEM),
        out_shape=jax.ShapeDtypeStruct((1, 1), jnp.float32),
        scratch_shapes=[
            pltpu.VMEM((2, _B, _WS), jnp.float32),
            pltpu.VMEM((2, _B, _WS), jnp.int32),
        ],
        compiler_params=pltpu.CompilerParams(
            dimension_semantics=("arbitrary", "arbitrary")),
    )(gt2, lmf, laf, u, u, xa, xb)
    return out[0, 0]


# speedup vs baseline: 1.0104x; 1.0104x over previous
import functools

import numpy as np
import jax
import jax.numpy as jnp
from jax import lax
from jax.experimental import pallas as pl
from jax.experimental.pallas import tpu as pltpu
from jax.experimental.pallas import tpu_sc as plsc

_K = 16
_B = 128
_V = 100000
_S = 24576
_VT = _V - _S
_W = 1024
_NV = (_VT + _W - 1) // _W
_WS = 2048
_NS = _S // _WS

_ROTS = ((13, 15, 26, 6), (17, 29, 16, 24))
_TINY = np.float32(np.finfo(np.float32).tiny)
_IMAX = np.int32(2**31 - 1)


def _np_threefry2x32(k0, k1, x0, x1):
    k0 = np.uint32(k0)
    k1 = np.uint32(k1)
    ks2 = np.uint32(k0 ^ k1 ^ np.uint32(0x1BD11BDA))
    sched = ((k1, ks2), (ks2, k0), (k0, k1), (k1, ks2), (ks2, k0))
    x0 = np.uint32(x0 + k0)
    x1 = np.uint32(x1 + k1)
    for i in range(5):
        for r in _ROTS[i % 2]:
            x0 = np.uint32(x0 + x1)
            x1 = np.uint32(np.uint32(x1 << np.uint32(r)) | np.uint32(x1 >> np.uint32(32 - r)))
            x1 = np.uint32(x1 ^ x0)
        a, b = sched[i]
        x0 = np.uint32(x0 + a)
        x1 = np.uint32(x1 + b + np.uint32(i + 1))
    return x0, x1


def _fold_in_keys():
    rows = []
    with np.errstate(over="ignore"):
        for seed in (101, 202):
            for j in range(_K):
                rows.append(_np_threefry2x32(0, seed, 0, j))
    return np.asarray(rows, dtype=np.uint32)


_KEYS = _fold_in_keys()
_KEYS_FLAT = np.zeros((2 * _K, 16), np.uint32)
_KEYS_FLAT[:, 0] = _KEYS[:, 0]
_KEYS_FLAT[:, 1] = _KEYS[:, 1]
_KEYS_FLAT = _KEYS_FLAT.reshape(-1)


def _sc_body(keys_hbm, u_hbm, keys_v, buf0, buf1, sem0, sem1):
    pltpu.sync_copy(keys_hbm, keys_v)
    wid = lax.axis_index("s") * 2 + lax.axis_index("c")
    kv = keys_v[pl.ds(wid * 16, 16)]
    k0 = kv[0]
    k1 = kv[1]
    ks2 = k0 ^ k1 ^ np.uint32(0x1BD11BDA)
    sched = ((k1, ks2), (ks2, k0), (k0, k1), (k1, ks2), (ks2, k0))
    lane = lax.iota(jnp.int32, 16).astype(jnp.uint32)

    def compute_row(b, buf):
        def chunk(i, carry2):
            base = (b * _V + _VT + i * 16).astype(jnp.uint32)
            flat = lane + base
            x1 = flat + k1
            x0 = x1 + k0
            first = True
            for ri in range(5):
                for r in _ROTS[ri % 2]:
                    if first:
                        first = False
                    else:
                        x0 = x0 + x1
                    x1 = ((x1 << r) | (x1 >> (32 - r))) ^ x0
                a, bb = sched[ri]
                x0 = x0 + a
                x1 = x1 + (bb + np.uint32(ri + 1))
            bits = x0 ^ x1
            fb = (bits >> 9) | np.uint32(0x3F800000)
            u = jnp.maximum(
                lax.bitcast_convert_type(fb, jnp.float32) - np.float32(1.0), _TINY)
            buf[pl.ds(i * 16, 16)] = u
            return carry2

        lax.fori_loop(0, _S // 16, chunk, 0)

    def rowpair(p, carry):
        b0 = 2 * p

        @pl.when(p > 0)
        def _():
            pltpu.make_async_copy(buf0, u_hbm.at[wid, b0 - 2], sem0).wait()

        compute_row(b0, buf0)
        pltpu.async_copy(buf0, u_hbm.at[wid, b0], sem0)

        @pl.when(p > 0)
        def _():
            pltpu.make_async_copy(buf1, u_hbm.at[wid, b0 - 1], sem1).wait()

        compute_row(b0 + 1, buf1)
        pltpu.async_copy(buf1, u_hbm.at[wid, b0 + 1], sem1)
        return carry

    lax.fori_loop(0, _B // 2, rowpair, 0)
    pltpu.make_async_copy(buf0, u_hbm.at[wid, _B - 2], sem0).wait()
    pltpu.make_async_copy(buf1, u_hbm.at[wid, _B - 1], sem1).wait()


def _sc_uniforms(keys):
    return pl.kernel(
        _sc_body,
        out_type=jax.ShapeDtypeStruct((2 * _K, _B, _S), jnp.float32),
        mesh=plsc.VectorSubcoreMesh(core_axis_name="c", subcore_axis_name="s"),
        scratch_types=[
            pltpu.VMEM((2 * _K * 16,), jnp.uint32),
            pltpu.VMEM((_S,), jnp.float32),
            pltpu.VMEM((_S,), jnp.float32),
            pltpu.SemaphoreType.DMA,
            pltpu.SemaphoreType.DMA,
        ],
    )(keys)


def _body1(keys_ref, a_ref, b_ref, lmf_ref, laf_ref, xm_ref, rm_ref, ci_ref):
    v = pl.program_id(0)
    v0 = (v * _W).astype(jnp.uint32)

    col = lax.broadcasted_iota(jnp.uint32, (_B, _W), 1) + v0
    rowv = lax.broadcasted_iota(jnp.uint32, (_B, _W), 0) * np.uint32(_V)
    flat = rowv + col
    valid = col < np.uint32(_VT)
    coli = lax.bitcast_convert_type(col, jnp.int32)

    @pl.when(v == 0)
    def _():
        rm_ref[...] = jnp.full((2 * _K, _B, _W), -jnp.inf, jnp.float32)
        ci_ref[...] = jnp.zeros((2 * _K, _B, _W), jnp.int32)

    xm_ref[0] = jnp.where(valid, a_ref[...], -jnp.inf)
    xm_ref[1] = jnp.where(valid, b_ref[...], -jnp.inf)

    def jbody(t, carry):
        k0 = keys_ref[t, 0]
        k1 = keys_ref[t, 1]
        ks2 = k0 ^ k1 ^ np.uint32(0x1BD11BDA)
        sched = ((k1, ks2), (ks2, k0), (k0, k1), (k1, ks2), (ks2, k0))
        x1 = flat + k1
        x0 = x1 + k0
        first = True
        for i in range(5):
            for r in _ROTS[i % 2]:
                if first:
                    first = False
                else:
                    x0 = x0 + x1
                x1 = ((x1 << r) | (x1 >> (32 - r))) ^ x0
            a, b = sched[i]
            x0 = x0 + a
            x1 = x1 + (b + np.uint32(i + 1))
        bits = x0 ^ x1
        fb = (bits >> 9) | np.uint32(0x3F800000)
        u = jnp.maximum(lax.bitcast_convert_type(fb, jnp.float32) - np.float32(1.0), _TINY)
        g = -jnp.log(-jnp.log(u))
        y = xm_ref[t // _K] + g
        old = rm_ref[t]
        upd = y > old
        rm_ref[t] = jnp.maximum(y, old)
        ci_ref[t] = jnp.where(upd, coli, ci_ref[t])
        return carry

    lax.fori_loop(0, 2 * _K, jbody, 0, unroll=4)

    @pl.when(v == _NV - 1)
    def _():
        for t in range(2 * _K):
            rmj = rm_ref[t]
            lm = jnp.max(rmj, axis=1, keepdims=True)
            cand = jnp.where(rmj == lm, ci_ref[t], _IMAX)
            la = jnp.min(cand, axis=1, keepdims=True)
            lmf_ref[:, t:t + 1] = lm
            laf_ref[:, t:t + 1] = la


def _body2(gt_ref, lmf_ref, laf_ref, ua_ref, ub_ref, xa_ref, xb_ref,
           out_ref, rm_ref, ci_ref):
    j = pl.program_id(0)
    v = pl.program_id(1)
    coli = (lax.broadcasted_iota(jnp.int32, (_B, _WS), 1)
            + (np.int32(_VT) + v * _WS))

    @pl.when(v == 0)
    def _():
        rm_ref[...] = jnp.full((2, _B, _WS), -jnp.inf, jnp.float32)
        ci_ref[...] = jnp.zeros((2, _B, _WS), jnp.int32)

    for d, (u_ref, x_ref) in enumerate(((ua_ref, xa_ref), (ub_ref, xb_ref))):
        u = u_ref[0]
        g = -jnp.log(-jnp.log(u))
        y = x_ref[...] + g
        old = rm_ref[d]
        upd = y > old
        rm_ref[d] = jnp.maximum(y, old)
        ci_ref[d] = jnp.where(upd, coli, ci_ref[d])

    @pl.when(v == _NS - 1)
    def _():
        la_final = []
        for d in (0, 1):
            t = d * _K + j
            rmj = rm_ref[d]
            lm2 = jnp.max(rmj, axis=1, keepdims=True)
            cand = jnp.where(rmj == lm2, ci_ref[d], _IMAX)
            la2 = jnp.min(cand, axis=1, keepdims=True)
            tcol = lax.broadcasted_iota(jnp.int32, (_B, 2 * _K), 1)
            sel = tcol == t
            lm1 = jnp.max(jnp.where(sel, lmf_ref[...], -jnp.inf),
                          axis=1, keepdims=True)
            la1 = jnp.max(jnp.where(sel, laf_ref[...], 0),
                          axis=1, keepdims=True)
            use2 = lm2 > lm1
            la_final.append(jnp.where(use2, la2, la1))
        res = la_final[0] + la_final[1]
        match = (res == gt_ref[...]).astype(jnp.float32)
        fmean = jnp.sum(match) * np.float32(1.0 / _B)
        lj = -jnp.log(fmean + np.float32(1e-8)) * np.float32(1.0 / _K)
        prev = jnp.where(j == 0, np.float32(0.0), out_ref[0, 0])
        out_ref[0, 0] = prev + lj


def kernel(gt, logits_a, logits_b):
    gt2 = gt.astype(jnp.int32).reshape(_B, 1)
    u = _sc_uniforms(jnp.asarray(_KEYS_FLAT))
    lmf, laf = pl.pallas_call(
        _body1,
        grid=(_NV,),
        in_specs=[
            pl.BlockSpec(memory_space=pltpu.SMEM),
            pl.BlockSpec((_B, _W), lambda v: (0, v)),
            pl.BlockSpec((_B, _W), lambda v: (0, v)),
        ],
        out_specs=[
            pl.BlockSpec((_B, 2 * _K), lambda v: (0, 0)),
            pl.BlockSpec((_B, 2 * _K), lambda v: (0, 0)),
        ],
        out_shape=[
            jax.ShapeDtypeStruct((_B, 2 * _K), jnp.float32),
            jax.ShapeDtypeStruct((_B, 2 * _K), jnp.int32),
        ],
        scratch_shapes=[
            pltpu.VMEM((2, _B, _W), jnp.float32),
            pltpu.VMEM((2 * _K, _B, _W), jnp.float32),
            pltpu.VMEM((2 * _K, _B, _W), jnp.int32),
        ],
        compiler_params=pltpu.CompilerParams(
            dimension_semantics=("arbitrary",)),
    )(jnp.asarray(_KEYS), logits_a, logits_b)

    xa = lax.slice(logits_a, (0, _VT), (_B, _V))
    xb = lax.slice(logits_b, (0, _VT), (_B, _V))
    out = pl.pallas_call(
        _body2,
        grid=(_K, _NS),
        in_specs=[
            pl.BlockSpec((_B, 1), lambda j, v: (0, 0)),
            pl.BlockSpec((_B, 2 * _K), lambda j, v: (0, 0)),
            pl.BlockSpec((_B, 2 * _K), lambda j, v: (0, 0)),
            pl.BlockSpec((1, _B, _WS), lambda j, v: (j, 0, v)),
            pl.BlockSpec((1, _B, _WS), lambda j, v: (j + _K, 0, v)),
            pl.BlockSpec((_B, _WS), lambda j, v: (0, v)),
            pl.BlockSpec((_B, _WS), lambda j, v: (0, v)),
        ],
        out_specs=pl.BlockSpec(memory_space=pltpu.SMEM),
        out_shape=jax.ShapeDtypeStruct((1, 1), jnp.float32),
        scratch_shapes=[
            pltpu.VMEM((2, _B, _WS), jnp.float32),
            pltpu.VMEM((2, _B, _WS), jnp.int32),
        ],
        compiler_params=pltpu.CompilerParams(
            dimension_semantics=("arbitrary", "arbitrary")),
    )(gt2, lmf, laf, u, u, xa, xb)
    return out[0, 0]


# speedup vs baseline: 1.0308x; 1.0202x over previous
import functools

import numpy as np
import jax
import jax.numpy as jnp
from jax import lax
from jax.experimental import pallas as pl
from jax.experimental.pallas import tpu as pltpu
from jax.experimental.pallas import tpu_sc as plsc

_K = 16
_B = 128
_V = 100000
_S = 26624
_VT = _V - _S
_W = 1024
_NV = (_VT + _W - 1) // _W
_WS = 2048
_NS = _S // _WS

_ROTS = ((13, 15, 26, 6), (17, 29, 16, 24))
_TINY = np.float32(np.finfo(np.float32).tiny)
_IMAX = np.int32(2**31 - 1)


def _np_threefry2x32(k0, k1, x0, x1):
    k0 = np.uint32(k0)
    k1 = np.uint32(k1)
    ks2 = np.uint32(k0 ^ k1 ^ np.uint32(0x1BD11BDA))
    sched = ((k1, ks2), (ks2, k0), (k0, k1), (k1, ks2), (ks2, k0))
    x0 = np.uint32(x0 + k0)
    x1 = np.uint32(x1 + k1)
    for i in range(5):
        for r in _ROTS[i % 2]:
            x0 = np.uint32(x0 + x1)
            x1 = np.uint32(np.uint32(x1 << np.uint32(r)) | np.uint32(x1 >> np.uint32(32 - r)))
            x1 = np.uint32(x1 ^ x0)
        a, b = sched[i]
        x0 = np.uint32(x0 + a)
        x1 = np.uint32(x1 + b + np.uint32(i + 1))
    return x0, x1


def _fold_in_keys():
    rows = []
    with np.errstate(over="ignore"):
        for seed in (101, 202):
            for j in range(_K):
                rows.append(_np_threefry2x32(0, seed, 0, j))
    return np.asarray(rows, dtype=np.uint32)


_KEYS = _fold_in_keys()
_KEYS_FLAT = np.zeros((2 * _K, 16), np.uint32)
_KEYS_FLAT[:, 0] = _KEYS[:, 0]
_KEYS_FLAT[:, 1] = _KEYS[:, 1]
_KEYS_FLAT = _KEYS_FLAT.reshape(-1)


def _sc_body(keys_hbm, u_hbm, keys_v, buf0, buf1, sem0, sem1):
    pltpu.sync_copy(keys_hbm, keys_v)
    wid = lax.axis_index("s") * 2 + lax.axis_index("c")
    kv = keys_v[pl.ds(wid * 16, 16)]
    k0 = kv[0]
    k1 = kv[1]
    ks2 = k0 ^ k1 ^ np.uint32(0x1BD11BDA)
    sched = ((k1, ks2), (ks2, k0), (k0, k1), (k1, ks2), (ks2, k0))
    lane = lax.iota(jnp.int32, 16).astype(jnp.uint32)

    def compute_row(b, buf):
        def chunk(i, carry2):
            base = (b * _V + _VT + i * 16).astype(jnp.uint32)
            flat = lane + base
            x1 = flat + k1
            x0 = x1 + k0
            first = True
            for ri in range(5):
                for r in _ROTS[ri % 2]:
                    if first:
                        first = False
                    else:
                        x0 = x0 + x1
                    x1 = ((x1 << r) | (x1 >> (32 - r))) ^ x0
                a, bb = sched[ri]
                x0 = x0 + a
                x1 = x1 + (bb + np.uint32(ri + 1))
            bits = x0 ^ x1
            fb = (bits >> 9) | np.uint32(0x3F800000)
            u = jnp.maximum(
                lax.bitcast_convert_type(fb, jnp.float32) - np.float32(1.0), _TINY)
            buf[pl.ds(i * 16, 16)] = u
            return carry2

        lax.fori_loop(0, _S // 16, chunk, 0)

    def rowpair(p, carry):
        b0 = 2 * p

        @pl.when(p > 0)
        def _():
            pltpu.make_async_copy(buf0, u_hbm.at[wid, b0 - 2], sem0).wait()

        compute_row(b0, buf0)
        pltpu.async_copy(buf0, u_hbm.at[wid, b0], sem0)

        @pl.when(p > 0)
        def _():
            pltpu.make_async_copy(buf1, u_hbm.at[wid, b0 - 1], sem1).wait()

        compute_row(b0 + 1, buf1)
        pltpu.async_copy(buf1, u_hbm.at[wid, b0 + 1], sem1)
        return carry

    lax.fori_loop(0, _B // 2, rowpair, 0)
    pltpu.make_async_copy(buf0, u_hbm.at[wid, _B - 2], sem0).wait()
    pltpu.make_async_copy(buf1, u_hbm.at[wid, _B - 1], sem1).wait()


def _sc_uniforms(keys):
    return pl.kernel(
        _sc_body,
        out_type=jax.ShapeDtypeStruct((2 * _K, _B, _S), jnp.float32),
        mesh=plsc.VectorSubcoreMesh(core_axis_name="c", subcore_axis_name="s"),
        scratch_types=[
            pltpu.VMEM((2 * _K * 16,), jnp.uint32),
            pltpu.VMEM((_S,), jnp.float32),
            pltpu.VMEM((_S,), jnp.float32),
            pltpu.SemaphoreType.DMA,
            pltpu.SemaphoreType.DMA,
        ],
    )(keys)


def _body1(keys_ref, a_ref, b_ref, lmf_ref, laf_ref, xm_ref, rm_ref, ci_ref):
    v = pl.program_id(0)
    v0 = (v * _W).astype(jnp.uint32)

    col = lax.broadcasted_iota(jnp.uint32, (_B, _W), 1) + v0
    rowv = lax.broadcasted_iota(jnp.uint32, (_B, _W), 0) * np.uint32(_V)
    flat = rowv + col
    valid = col < np.uint32(_VT)
    coli = lax.bitcast_convert_type(col, jnp.int32)

    @pl.when(v == 0)
    def _():
        rm_ref[...] = jnp.full((2 * _K, _B, _W), -jnp.inf, jnp.float32)
        ci_ref[...] = jnp.zeros((2 * _K, _B, _W), jnp.int32)

    xm_ref[0] = jnp.where(valid, a_ref[...], -jnp.inf)
    xm_ref[1] = jnp.where(valid, b_ref[...], -jnp.inf)

    def jbody(t, carry):
        k0 = keys_ref[t, 0]
        k1 = keys_ref[t, 1]
        ks2 = k0 ^ k1 ^ np.uint32(0x1BD11BDA)
        sched = ((k1, ks2), (ks2, k0), (k0, k1), (k1, ks2), (ks2, k0))
        x1 = flat + k1
        x0 = x1 + k0
        first = True
        for i in range(5):
            for r in _ROTS[i % 2]:
                if first:
                    first = False
                else:
                    x0 = x0 + x1
                x1 = ((x1 << r) | (x1 >> (32 - r))) ^ x0
            a, b = sched[i]
            x0 = x0 + a
            x1 = x1 + (b + np.uint32(i + 1))
        bits = x0 ^ x1
        fb = (bits >> 9) | np.uint32(0x3F800000)
        u = jnp.maximum(lax.bitcast_convert_type(fb, jnp.float32) - np.float32(1.0), _TINY)
        g = -jnp.log(-jnp.log(u))
        y = xm_ref[t // _K] + g
        old = rm_ref[t]
        upd = y > old
        rm_ref[t] = jnp.maximum(y, old)
        ci_ref[t] = jnp.where(upd, coli, ci_ref[t])
        return carry

    lax.fori_loop(0, 2 * _K, jbody, 0, unroll=4)

    @pl.when(v == _NV - 1)
    def _():
        for t in range(2 * _K):
            rmj = rm_ref[t]
            lm = jnp.max(rmj, axis=1, keepdims=True)
            cand = jnp.where(rmj == lm, ci_ref[t], _IMAX)
            la = jnp.min(cand, axis=1, keepdims=True)
            lmf_ref[:, t:t + 1] = lm
            laf_ref[:, t:t + 1] = la


def _body2(gt_ref, lmf_ref, laf_ref, ua_ref, ub_ref, xa_ref, xb_ref,
           out_ref, rm_ref, ci_ref):
    j = pl.program_id(0)
    v = pl.program_id(1)
    coli = (lax.broadcasted_iota(jnp.int32, (_B, _WS), 1)
            + (np.int32(_VT) + v * _WS))

    @pl.when(v == 0)
    def _():
        rm_ref[...] = jnp.full((2, _B, _WS), -jnp.inf, jnp.float32)
        ci_ref[...] = jnp.zeros((2, _B, _WS), jnp.int32)

    for d, (u_ref, x_ref) in enumerate(((ua_ref, xa_ref), (ub_ref, xb_ref))):
        u = u_ref[0]
        g = -jnp.log(-jnp.log(u))
        y = x_ref[...] + g
        old = rm_ref[d]
        upd = y > old
        rm_ref[d] = jnp.maximum(y, old)
        ci_ref[d] = jnp.where(upd, coli, ci_ref[d])

    @pl.when(v == _NS - 1)
    def _():
        la_final = []
        for d in (0, 1):
            t = d * _K + j
            rmj = rm_ref[d]
            lm2 = jnp.max(rmj, axis=1, keepdims=True)
            cand = jnp.where(rmj == lm2, ci_ref[d], _IMAX)
            la2 = jnp.min(cand, axis=1, keepdims=True)
            tcol = lax.broadcasted_iota(jnp.int32, (_B, 2 * _K), 1)
            sel = tcol == t
            lm1 = jnp.max(jnp.where(sel, lmf_ref[...], -jnp.inf),
                          axis=1, keepdims=True)
            la1 = jnp.max(jnp.where(sel, laf_ref[...], 0),
                          axis=1, keepdims=True)
            use2 = lm2 > lm1
            la_final.append(jnp.where(use2, la2, la1))
        res = la_final[0] + la_final[1]
        match = (res == gt_ref[...]).astype(jnp.float32)
        fmean = jnp.sum(match) * np.float32(1.0 / _B)
        lj = -jnp.log(fmean + np.float32(1e-8)) * np.float32(1.0 / _K)
        prev = jnp.where(j == 0, np.float32(0.0), out_ref[0, 0])
        out_ref[0, 0] = prev + lj


def kernel(gt, logits_a, logits_b):
    gt2 = gt.astype(jnp.int32).reshape(_B, 1)
    u = _sc_uniforms(jnp.asarray(_KEYS_FLAT))
    lmf, laf = pl.pallas_call(
        _body1,
        grid=(_NV,),
        in_specs=[
            pl.BlockSpec(memory_space=pltpu.SMEM),
            pl.BlockSpec((_B, _W), lambda v: (0, v)),
            pl.BlockSpec((_B, _W), lambda v: (0, v)),
        ],
        out_specs=[
            pl.BlockSpec((_B, 2 * _K), lambda v: (0, 0)),
            pl.BlockSpec((_B, 2 * _K), lambda v: (0, 0)),
        ],
        out_shape=[
            jax.ShapeDtypeStruct((_B, 2 * _K), jnp.float32),
            jax.ShapeDtypeStruct((_B, 2 * _K), jnp.int32),
        ],
        scratch_shapes=[
            pltpu.VMEM((2, _B, _W), jnp.float32),
            pltpu.VMEM((2 * _K, _B, _W), jnp.float32),
            pltpu.VMEM((2 * _K, _B, _W), jnp.int32),
        ],
        compiler_params=pltpu.CompilerParams(
            dimension_semantics=("arbitrary",)),
    )(jnp.asarray(_KEYS), logits_a, logits_b)

    xa = lax.slice(logits_a, (0, _VT), (_B, _V))
    xb = lax.slice(logits_b, (0, _VT), (_B, _V))
    out = pl.pallas_call(
        _body2,
        grid=(_K, _NS),
        in_specs=[
            pl.BlockSpec((_B, 1), lambda j, v: (0, 0)),
            pl.BlockSpec((_B, 2 * _K), lambda j, v: (0, 0)),
            pl.BlockSpec((_B, 2 * _K), lambda j, v: (0, 0)),
            pl.BlockSpec((1, _B, _WS), lambda j, v: (j, 0, v)),
            pl.BlockSpec((1, _B, _WS), lambda j, v: (j + _K, 0, v)),
            pl.BlockSpec((_B, _WS), lambda j, v: (0, v)),
            pl.BlockSpec((_B, _WS), lambda j, v: (0, v)),
        ],
        out_specs=pl.BlockSpec(memory_space=pltpu.SMEM),
        out_shape=jax.ShapeDtypeStruct((1, 1), jnp.float32),
        scratch_shapes=[
            pltpu.VMEM((2, _B, _WS), jnp.float32),
            pltpu.VMEM((2, _B, _WS), jnp.int32),
        ],
        compiler_params=pltpu.CompilerParams(
            dimension_semantics=("arbitrary", "arbitrary")),
    )(gt2, lmf, laf, u, u, xa, xb)
    return out[0, 0]


# speedup vs baseline: 1.0315x; 1.0007x over previous
import functools

import numpy as np
import jax
import jax.numpy as jnp
from jax import lax
from jax.experimental import pallas as pl
from jax.experimental.pallas import tpu as pltpu
from jax.experimental.pallas import tpu_sc as plsc

_K = 16
_B = 128
_V = 100000
_S = 26624
_VT = _V - _S
_W = 1024
_NV = (_VT + _W - 1) // _W
_WS = 2048
_NS = _S // _WS

_ROTS = ((13, 15, 26, 6), (17, 29, 16, 24))
_TINY = np.float32(np.finfo(np.float32).tiny)
_IMAX = np.int32(2**31 - 1)


def _np_threefry2x32(k0, k1, x0, x1):
    k0 = np.uint32(k0)
    k1 = np.uint32(k1)
    ks2 = np.uint32(k0 ^ k1 ^ np.uint32(0x1BD11BDA))
    sched = ((k1, ks2), (ks2, k0), (k0, k1), (k1, ks2), (ks2, k0))
    x0 = np.uint32(x0 + k0)
    x1 = np.uint32(x1 + k1)
    for i in range(5):
        for r in _ROTS[i % 2]:
            x0 = np.uint32(x0 + x1)
            x1 = np.uint32(np.uint32(x1 << np.uint32(r)) | np.uint32(x1 >> np.uint32(32 - r)))
            x1 = np.uint32(x1 ^ x0)
        a, b = sched[i]
        x0 = np.uint32(x0 + a)
        x1 = np.uint32(x1 + b + np.uint32(i + 1))
    return x0, x1


def _fold_in_keys():
    rows = []
    with np.errstate(over="ignore"):
        for seed in (101, 202):
            for j in range(_K):
                rows.append(_np_threefry2x32(0, seed, 0, j))
    return np.asarray(rows, dtype=np.uint32)


_KEYS = _fold_in_keys()
_KEYS_FLAT = np.zeros((2 * _K, 16), np.uint32)
_KEYS_FLAT[:, 0] = _KEYS[:, 0]
_KEYS_FLAT[:, 1] = _KEYS[:, 1]
_KEYS_FLAT = _KEYS_FLAT.reshape(-1)


def _sc_body(keys_hbm, u_hbm, keys_v, buf0, buf1, sem0, sem1):
    pltpu.sync_copy(keys_hbm, keys_v)
    wid = lax.axis_index("s") * 2 + lax.axis_index("c")
    kv = keys_v[pl.ds(wid * 16, 16)]
    k0 = kv[0]
    k1 = kv[1]
    ks2 = k0 ^ k1 ^ np.uint32(0x1BD11BDA)
    sched = ((k1, ks2), (ks2, k0), (k0, k1), (k1, ks2), (ks2, k0))
    lane = lax.iota(jnp.int32, 16).astype(jnp.uint32)

    def compute_row(b, buf):
        def chunk(i, carry2):
            base = (b * _V + _VT + i * 16).astype(jnp.uint32)
            flat = lane + base
            x1 = flat + k1
            x0 = x1 + k0
            first = True
            for ri in range(5):
                for r in _ROTS[ri % 2]:
                    if first:
                        first = False
                    else:
                        x0 = x0 + x1
                    x1 = ((x1 << r) | (x1 >> (32 - r))) ^ x0
                a, bb = sched[ri]
                x0 = x0 + a
                x1 = x1 + (bb + np.uint32(ri + 1))
            bits = x0 ^ x1
            fb = (bits >> 9) | np.uint32(0x3F800000)
            u = jnp.maximum(
                lax.bitcast_convert_type(fb, jnp.float32) - np.float32(1.0), _TINY)
            buf[pl.ds(i * 16, 16)] = u
            return carry2

        lax.fori_loop(0, _S // 16, chunk, 0)

    def rowpair(p, carry):
        b0 = 2 * p

        @pl.when(p > 0)
        def _():
            pltpu.make_async_copy(buf0, u_hbm.at[wid, b0 - 2], sem0).wait()

        compute_row(b0, buf0)
        pltpu.async_copy(buf0, u_hbm.at[wid, b0], sem0)

        @pl.when(p > 0)
        def _():
            pltpu.make_async_copy(buf1, u_hbm.at[wid, b0 - 1], sem1).wait()

        compute_row(b0 + 1, buf1)
        pltpu.async_copy(buf1, u_hbm.at[wid, b0 + 1], sem1)
        return carry

    lax.fori_loop(0, _B // 2, rowpair, 0)
    pltpu.make_async_copy(buf0, u_hbm.at[wid, _B - 2], sem0).wait()
    pltpu.make_async_copy(buf1, u_hbm.at[wid, _B - 1], sem1).wait()


def _sc_uniforms(keys):
    return pl.kernel(
        _sc_body,
        out_type=jax.ShapeDtypeStruct((2 * _K, _B, _S), jnp.float32),
        mesh=plsc.VectorSubcoreMesh(core_axis_name="c", subcore_axis_name="s"),
        scratch_types=[
            pltpu.VMEM((2 * _K * 16,), jnp.uint32),
            pltpu.VMEM((_S,), jnp.float32),
            pltpu.VMEM((_S,), jnp.float32),
            pltpu.SemaphoreType.DMA,
            pltpu.SemaphoreType.DMA,
        ],
    )(keys)


def _body1(keys_ref, a_ref, b_ref, lmf_ref, laf_ref, xm_ref, rm_ref, ci_ref):
    v = pl.program_id(0)
    v0 = (v * _W).astype(jnp.uint32)

    col = lax.broadcasted_iota(jnp.uint32, (_B, _W), 1) + v0
    rowv = lax.broadcasted_iota(jnp.uint32, (_B, _W), 0) * np.uint32(_V)
    flat = rowv + col
    valid = col < np.uint32(_VT)
    coli = lax.bitcast_convert_type(col, jnp.int32)

    @pl.when(v == 0)
    def _():
        rm_ref[...] = jnp.full((2 * _K, _B, _W), -jnp.inf, jnp.float32)
        ci_ref[...] = jnp.zeros((2 * _K, _B, _W), jnp.int32)

    xm_ref[0] = jnp.where(valid, a_ref[...], -jnp.inf)
    xm_ref[1] = jnp.where(valid, b_ref[...], -jnp.inf)

    def jbody(t, carry):
        k0 = keys_ref[t, 0]
        k1 = keys_ref[t, 1]
        ks2 = k0 ^ k1 ^ np.uint32(0x1BD11BDA)
        sched = ((k1, ks2), (ks2, k0), (k0, k1), (k1, ks2), (ks2, k0))
        x1 = flat + k1
        x0 = x1 + k0
        first = True
        for i in range(5):
            for r in _ROTS[i % 2]:
                if first:
                    first = False
                else:
                    x0 = x0 + x1
                x1 = ((x1 << r) | (x1 >> (32 - r))) ^ x0
            a, b = sched[i]
            x0 = x0 + a
            x1 = x1 + (b + np.uint32(i + 1))
        bits = x0 ^ x1
        fb = (bits >> 9) | np.uint32(0x3F800000)
        u = jnp.maximum(lax.bitcast_convert_type(fb, jnp.float32) - np.float32(1.0), _TINY)
        g = -jnp.log(-jnp.log(u))
        y = xm_ref[t // _K] + g
        old = rm_ref[t]
        upd = y > old
        rm_ref[t] = jnp.maximum(y, old)
        ci_ref[t] = jnp.where(upd, coli, ci_ref[t])
        return carry

    lax.fori_loop(0, 2 * _K, jbody, 0, unroll=8)

    @pl.when(v == _NV - 1)
    def _():
        for t in range(2 * _K):
            rmj = rm_ref[t]
            lm = jnp.max(rmj, axis=1, keepdims=True)
            cand = jnp.where(rmj == lm, ci_ref[t], _IMAX)
            la = jnp.min(cand, axis=1, keepdims=True)
            lmf_ref[:, t:t + 1] = lm
            laf_ref[:, t:t + 1] = la


def _body2(gt_ref, lmf_ref, laf_ref, ua_ref, ub_ref, xa_ref, xb_ref,
           out_ref, rm_ref, ci_ref):
    j = pl.program_id(0)
    v = pl.program_id(1)
    coli = (lax.broadcasted_iota(jnp.int32, (_B, _WS), 1)
            + (np.int32(_VT) + v * _WS))

    @pl.when(v == 0)
    def _():
        rm_ref[...] = jnp.full((2, _B, _WS), -jnp.inf, jnp.float32)
        ci_ref[...] = jnp.zeros((2, _B, _WS), jnp.int32)

    for d, (u_ref, x_ref) in enumerate(((ua_ref, xa_ref), (ub_ref, xb_ref))):
        u = u_ref[0]
        g = -jnp.log(-jnp.log(u))
        y = x_ref[...] + g
        old = rm_ref[d]
        upd = y > old
        rm_ref[d] = jnp.maximum(y, old)
        ci_ref[d] = jnp.where(upd, coli, ci_ref[d])

    @pl.when(v == _NS - 1)
    def _():
        la_final = []
        for d in (0, 1):
            t = d * _K + j
            rmj = rm_ref[d]
            lm2 = jnp.max(rmj, axis=1, keepdims=True)
            cand = jnp.where(rmj == lm2, ci_ref[d], _IMAX)
            la2 = jnp.min(cand, axis=1, keepdims=True)
            tcol = lax.broadcasted_iota(jnp.int32, (_B, 2 * _K), 1)
            sel = tcol == t
            lm1 = jnp.max(jnp.where(sel, lmf_ref[...], -jnp.inf),
                          axis=1, keepdims=True)
            la1 = jnp.max(jnp.where(sel, laf_ref[...], 0),
                          axis=1, keepdims=True)
            use2 = lm2 > lm1
            la_final.append(jnp.where(use2, la2, la1))
        res = la_final[0] + la_final[1]
        match = (res == gt_ref[...]).astype(jnp.float32)
        fmean = jnp.sum(match) * np.float32(1.0 / _B)
        lj = -jnp.log(fmean + np.float32(1e-8)) * np.float32(1.0 / _K)
        prev = jnp.where(j == 0, np.float32(0.0), out_ref[0, 0])
        out_ref[0, 0] = prev + lj


def kernel(gt, logits_a, logits_b):
    gt2 = gt.astype(jnp.int32).reshape(_B, 1)
    u = _sc_uniforms(jnp.asarray(_KEYS_FLAT))
    lmf, laf = pl.pallas_call(
        _body1,
        grid=(_NV,),
        in_specs=[
            pl.BlockSpec(memory_space=pltpu.SMEM),
            pl.BlockSpec((_B, _W), lambda v: (0, v)),
            pl.BlockSpec((_B, _W), lambda v: (0, v)),
        ],
        out_specs=[
            pl.BlockSpec((_B, 2 * _K), lambda v: (0, 0)),
            pl.BlockSpec((_B, 2 * _K), lambda v: (0, 0)),
        ],
        out_shape=[
            jax.ShapeDtypeStruct((_B, 2 * _K), jnp.float32),
            jax.ShapeDtypeStruct((_B, 2 * _K), jnp.int32),
        ],
        scratch_shapes=[
            pltpu.VMEM((2, _B, _W), jnp.float32),
            pltpu.VMEM((2 * _K, _B, _W), jnp.float32),
            pltpu.VMEM((2 * _K, _B, _W), jnp.int32),
        ],
        compiler_params=pltpu.CompilerParams(
            dimension_semantics=("arbitrary",)),
    )(jnp.asarray(_KEYS), logits_a, logits_b)

    xa = lax.slice(logits_a, (0, _VT), (_B, _V))
    xb = lax.slice(logits_b, (0, _VT), (_B, _V))
    out = pl.pallas_call(
        _body2,
        grid=(_K, _NS),
        in_specs=[
            pl.BlockSpec((_B, 1), lambda j, v: (0, 0)),
            pl.BlockSpec((_B, 2 * _K), lambda j, v: (0, 0)),
            pl.BlockSpec((_B, 2 * _K), lambda j, v: (0, 0)),
            pl.BlockSpec((1, _B, _WS), lambda j, v: (j, 0, v)),
            pl.BlockSpec((1, _B, _WS), lambda j, v: (j + _K, 0, v)),
            pl.BlockSpec((_B, _WS), lambda j, v: (0, v)),
            pl.BlockSpec((_B, _WS), lambda j, v: (0, v)),
        ],
        out_specs=pl.BlockSpec(memory_space=pltpu.SMEM),
        out_shape=jax.ShapeDtypeStruct((1, 1), jnp.float32),
        scratch_shapes=[
            pltpu.VMEM((2, _B, _WS), jnp.float32),
            pltpu.VMEM((2, _B, _WS), jnp.int32),
        ],
        compiler_params=pltpu.CompilerParams(
            dimension_semantics=("arbitrary", "arbitrary")),
    )(gt2, lmf, laf, u, u, xa, xb)
    return out[0, 0]
